# baseline (speedup 1.0000x reference)
import jax
import jax.numpy as jnp
from jax import lax
from jax.experimental import pallas as pl
from jax.experimental.pallas import tpu as pltpu
from jax.experimental.pallas import tpu_sc as plsc

N = 10000
NPAD = 10240
E = 320000
D = 128
NC = 2
NS = 16
NW = NC * NS
EW = E // NW
K = 80
NCH = EW // K
SL = NPAD // NS
NBUF = 4
NIB = 2 * NBUF
NMAIN = (NCH // NIB) * NIB
AG = 5
RB = 10


def _mesh():
    return plsc.VectorSubcoreMesh(
        core_axis_name="c", subcore_axis_name="s", num_cores=NC, num_subcores=NS
    )


def _fill_ones(ones):
    for i in range(K // 16):
        ones[pl.ds(i * 16, 16)] = jnp.ones((16,), jnp.float32)


def _deg_body(src3, dst3, zeros1, degp, sidx, didx, ones, sh_out, sh_in, sem):
    c = lax.axis_index("c")
    s = lax.axis_index("s")
    wid = s * NC + c
    pltpu.sync_copy(zeros1, sh_out.at[pl.ds(s * SL, SL)])
    pltpu.sync_copy(zeros1, sh_in.at[pl.ds(s * SL, SL)])
    _fill_ones(ones)
    pltpu.sync_copy(src3.at[wid], sidx)
    pltpu.sync_copy(dst3.at[wid], didx)
    plsc.subcore_barrier()

    def group(g, carry):
        for t in range(AG):
            j = g * AG + t
            pltpu.async_copy(ones, sh_out.at[sidx.at[j]], sem, add=True)
            pltpu.async_copy(ones, sh_in.at[didx.at[j]], sem, add=True)
        for t in range(AG):
            j = g * AG + t
            pltpu.make_async_copy(ones, sh_out.at[sidx.at[j]], sem).wait()
            pltpu.make_async_copy(ones, sh_in.at[didx.at[j]], sem).wait()
        return carry

    lax.fori_loop(0, NCH // AG, group, 0)
    plsc.subcore_barrier()
    pltpu.sync_copy(sh_out.at[pl.ds(s * SL, SL)], degp.at[0, c, pl.ds(s * SL, SL)])
    pltpu.sync_copy(sh_in.at[pl.ds(s * SL, SL)], degp.at[1, c, pl.ds(s * SL, SL)])


def _spmm_body(table, src3, dst3, aggp, sbuf, dbuf, rows, sh_acc,
               isem, gsem, ssem):
    c = lax.axis_index("c")
    s = lax.axis_index("s")
    wid = s * NC + c
    zv = jnp.zeros((16,), jnp.float32)

    def zrow(i, carry):
        for kk in range(D // 16):
            rows[0, i, pl.ds(kk * 16, 16)] = zv
        return carry

    lax.fori_loop(0, K, zrow, 0)
    for r in range(SL // K):
        pltpu.sync_copy(rows.at[0], sh_acc.at[pl.ds(s * SL + r * K, K)])
    plsc.subcore_barrier()

    def fire_idx(j, slot):
        pltpu.async_copy(src3.at[wid, j], sbuf.at[slot], isem.at[slot])
        pltpu.async_copy(dst3.at[wid, j], dbuf.at[slot], isem.at[slot])

    def wait_idx(slot):
        pltpu.make_async_copy(src3.at[wid, 0], sbuf.at[slot], isem.at[slot]).wait()
        pltpu.make_async_copy(src3.at[wid, 0], dbuf.at[slot], isem.at[slot]).wait()

    def fire_gather(slot, b):
        pltpu.async_copy(table.at[sbuf.at[slot]], rows.at[b], gsem.at[b])

    def wait_gather(b):
        pltpu.make_async_copy(table.at[pl.ds(0, K)], rows.at[b], gsem.at[b]).wait()

    def fire_scatter(ib, b):
        pltpu.async_copy(rows.at[b], sh_acc.at[dbuf.at[ib]], ssem.at[b], add=True)

    def wait_scatter(ib, b):
        pltpu.make_async_copy(rows.at[b], sh_acc.at[dbuf.at[ib]], ssem.at[b]).wait()

    for b in range(NBUF):
        fire_idx(b, b)
    for b in range(NBUF):
        wait_idx(b)
        fire_gather(b, b)
        fire_idx(b + NBUF, b + NBUF)

    def outer(g, carry):
        for p in range(NIB // 2):
            ib0, ib1 = 2 * p, 2 * p + 1
            b0, b1 = ib0 % NBUF, ib1 % NBUF
            j0 = g * NIB + ib0
            j1 = j0 + 1
            wait_gather(b0)
            wait_gather(b1)
            fire_scatter(ib0, b0)
            fire_scatter(ib1, b1)
            wait_scatter(ib0, b0)
            wait_scatter(ib1, b1)

            @pl.when(j0 + NIB < NCH)
            def _refill0():
                fire_idx(j0 + NIB, ib0)

            @pl.when(j1 + NIB < NCH)
            def _refill1():
                fire_idx(j1 + NIB, ib1)

            wait_idx((ib0 + NBUF) % NIB)
            fire_gather((ib0 + NBUF) % NIB, b0)
            wait_idx((ib1 + NBUF) % NIB)
            fire_gather((ib1 + NBUF) % NIB, b1)
        return carry

    lax.fori_loop(0, NMAIN // NIB, outer, 0)

    for t in range(NCH - NMAIN):
        j = NMAIN + t
        ib = j % NIB
        b = ib % NBUF
        wait_gather(b)
        pltpu.sync_copy(rows.at[b], sh_acc.at[dbuf.at[ib]], add=True)
        nxt = j + NBUF
        if nxt < NCH:
            nslot = nxt % NIB
            wait_idx(nslot)
            fire_gather(nslot, b)

    plsc.subcore_barrier()
    pltpu.sync_copy(sh_acc.at[pl.ds(s * SL, SL)], aggp.at[c, pl.ds(s * SL, SL)])


def _src_norm_body(f_ref, d_ref, o_ref):
    deg = d_ref[0, :, :] + d_ref[1, :, :]
    norm = lax.rsqrt(jnp.maximum(deg, 1.0))
    o_ref[...] = f_ref[...] * norm


def _out_body(a_ref, w_ref, d_ref, b_ref, o_ref):
    agg = a_ref[0, :, :] + a_ref[1, :, :]
    rst = jnp.dot(agg, w_ref[...], preferred_element_type=jnp.float32)
    deg = d_ref[0, :, :] + d_ref[1, :, :]
    norm = lax.rsqrt(jnp.maximum(deg, 1.0))
    o_ref[...] = rst * norm + b_ref[...]


@jax.jit
def kernel(feat, edge_index, weight, bias):
    src3 = edge_index[0].astype(jnp.int32).reshape(NW, NCH, K)
    dst3 = edge_index[1].astype(jnp.int32).reshape(NW, NCH, K)
    zeros1 = jnp.zeros((SL,), jnp.float32)

    degp = pl.kernel(
        _deg_body,
        out_type=jax.ShapeDtypeStruct((2, NC, NPAD), jnp.float32),
        mesh=_mesh(),
        scratch_types=[
            pltpu.VMEM((NCH, K), jnp.int32),
            pltpu.VMEM((NCH, K), jnp.int32),
            pltpu.VMEM((K,), jnp.float32),
            pltpu.VMEM_SHARED((NPAD,), jnp.float32),
            pltpu.VMEM_SHARED((NPAD,), jnp.float32),
            pltpu.SemaphoreType.DMA,
        ],
    )(src3, dst3, zeros1)

    nb = N // RB
    feat_src = pl.pallas_call(
        _src_norm_body,
        grid=(RB,),
        in_specs=[
            pl.BlockSpec((nb, D), lambda i: (i, 0)),
            pl.BlockSpec((NC, nb, 1), lambda i: (0, i, 0)),
        ],
        out_specs=pl.BlockSpec((nb, D), lambda i: (i, 0)),
        out_shape=jax.ShapeDtypeStruct((N, D), jnp.float32),
    )(feat, degp[0][:, :, None])

    aggp = pl.kernel(
        _spmm_body,
        out_type=jax.ShapeDtypeStruct((NC, NPAD, D), jnp.float32),
        mesh=_mesh(),
        scratch_types=[
            pltpu.VMEM((NIB, K), jnp.int32),
            pltpu.VMEM((NIB, K), jnp.int32),
            pltpu.VMEM((NBUF, K, D), jnp.float32),
            pltpu.VMEM_SHARED((NPAD, D), jnp.float32),
            pltpu.SemaphoreType.DMA((NIB,)),
            pltpu.SemaphoreType.DMA((NBUF,)),
            pltpu.SemaphoreType.DMA((NBUF,)),
        ],
    )(feat_src, src3, dst3)

    out = pl.pallas_call(
        _out_body,
        grid=(RB,),
        in_specs=[
            pl.BlockSpec((NC, nb, D), lambda i: (0, i, 0)),
            pl.BlockSpec((D, D), lambda i: (0, 0)),
            pl.BlockSpec((NC, nb, 1), lambda i: (0, i, 0)),
            pl.BlockSpec((1, D), lambda i: (0, 0)),
        ],
        out_specs=pl.BlockSpec((nb, D), lambda i: (i, 0)),
        out_shape=jax.ShapeDtypeStruct((N, D), jnp.float32),
    )(aggp, weight, degp[1][:, :, None], bias.reshape(1, D))
    return out


# speedup vs baseline: 1.1083x; 1.1083x over previous
import jax
import jax.numpy as jnp
from jax import lax
from jax.experimental import pallas as pl
from jax.experimental.pallas import tpu as pltpu
from jax.experimental.pallas import tpu_sc as plsc

N = 10000
NPAD = 10240
E = 320000
D = 128
NC = 2
NS = 16
NW = NC * NS
EW = E // NW
K = 80
NCH = EW // K
SL = NPAD // NS
NBUF = 4
NIB = 2 * NBUF
NMAIN = (NCH // NIB) * NIB
AG = 5
RB = 10


def _mesh():
    return plsc.VectorSubcoreMesh(
        core_axis_name="c", subcore_axis_name="s", num_cores=NC, num_subcores=NS
    )


def _fill_ones(ones):
    for i in range(K // 16):
        ones[pl.ds(i * 16, 16)] = jnp.ones((16,), jnp.float32)


def _deg_body(src3, dst3, zeros1, degp, sidx, didx, ones, sh_out, sh_in, sem):
    c = lax.axis_index("c")
    s = lax.axis_index("s")
    wid = s * NC + c
    pltpu.sync_copy(zeros1, sh_out.at[pl.ds(s * SL, SL)])
    pltpu.sync_copy(zeros1, sh_in.at[pl.ds(s * SL, SL)])
    _fill_ones(ones)
    pltpu.sync_copy(src3.at[wid], sidx)
    pltpu.sync_copy(dst3.at[wid], didx)
    plsc.subcore_barrier()

    def group(g, carry):
        for t in range(AG):
            j = g * AG + t
            pltpu.async_copy(ones, sh_out.at[sidx.at[j]], sem, add=True)
            pltpu.async_copy(ones, sh_in.at[didx.at[j]], sem, add=True)
        for t in range(AG):
            j = g * AG + t
            pltpu.make_async_copy(ones, sh_out.at[sidx.at[j]], sem).wait()
            pltpu.make_async_copy(ones, sh_in.at[didx.at[j]], sem).wait()
        return carry

    lax.fori_loop(0, NCH // AG, group, 0)
    plsc.subcore_barrier()
    pltpu.sync_copy(sh_out.at[pl.ds(s * SL, SL)], degp.at[0, c, pl.ds(s * SL, SL)])
    pltpu.sync_copy(sh_in.at[pl.ds(s * SL, SL)], degp.at[1, c, pl.ds(s * SL, SL)])


def _spmm_body(table, src3, dst3, aggp, sbuf, dbuf, rows, sh_acc,
               isem, gsem):
    c = lax.axis_index("c")
    s = lax.axis_index("s")
    wid = s * NC + c
    zv = jnp.zeros((16,), jnp.float32)

    def zrow(i, carry):
        for kk in range(D // 16):
            rows[0, i, pl.ds(kk * 16, 16)] = zv
        return carry

    lax.fori_loop(0, K, zrow, 0)
    for r in range(SL // K):
        pltpu.sync_copy(rows.at[0], sh_acc.at[pl.ds(s * SL + r * K, K)])
    plsc.subcore_barrier()

    def fire_idx(j, slot):
        pltpu.async_copy(src3.at[wid, j], sbuf.at[slot], isem.at[slot])
        pltpu.async_copy(dst3.at[wid, j], dbuf.at[slot], isem.at[slot])

    def wait_idx(slot):
        pltpu.make_async_copy(src3.at[wid, 0], sbuf.at[slot], isem.at[slot]).wait()
        pltpu.make_async_copy(src3.at[wid, 0], dbuf.at[slot], isem.at[slot]).wait()

    def fire_gather(slot, b):
        pltpu.async_copy(table.at[sbuf.at[slot]], rows.at[b], gsem.at[b])

    def wait_gather(b):
        pltpu.make_async_copy(table.at[pl.ds(0, K)], rows.at[b], gsem.at[b]).wait()


    for b in range(NBUF):
        fire_idx(b, b)
    for b in range(NBUF):
        wait_idx(b)
        fire_gather(b, b)
        fire_idx(b + NBUF, b + NBUF)

    def outer(g, carry):
        for ib in range(NIB):
            b = ib % NBUF
            j = g * NIB + ib
            wait_gather(b)
            pltpu.sync_copy(rows.at[b], sh_acc.at[dbuf.at[ib]], add=True)

            @pl.when(j + NIB < NCH)
            def _refill():
                fire_idx(j + NIB, ib)

            nslot = (ib + NBUF) % NIB
            wait_idx(nslot)
            fire_gather(nslot, b)
        return carry

    lax.fori_loop(0, NMAIN // NIB, outer, 0)

    for t in range(NCH - NMAIN):
        j = NMAIN + t
        ib = j % NIB
        b = ib % NBUF
        wait_gather(b)
        pltpu.sync_copy(rows.at[b], sh_acc.at[dbuf.at[ib]], add=True)
        nxt = j + NBUF
        if nxt < NCH:
            nslot = nxt % NIB
            wait_idx(nslot)
            fire_gather(nslot, b)

    plsc.subcore_barrier()
    pltpu.sync_copy(sh_acc.at[pl.ds(s * SL, SL)], aggp.at[c, pl.ds(s * SL, SL)])


def _src_norm_body(f_ref, d_ref, o_ref):
    deg = d_ref[0, :, :] + d_ref[1, :, :]
    norm = lax.rsqrt(jnp.maximum(deg, 1.0))
    o_ref[...] = f_ref[...] * norm


def _out_body(a_ref, w_ref, d_ref, b_ref, o_ref):
    agg = a_ref[0, :, :] + a_ref[1, :, :]
    rst = jnp.dot(agg, w_ref[...], preferred_element_type=jnp.float32)
    deg = d_ref[0, :, :] + d_ref[1, :, :]
    norm = lax.rsqrt(jnp.maximum(deg, 1.0))
    o_ref[...] = rst * norm + b_ref[...]


@jax.jit
def kernel(feat, edge_index, weight, bias):
    src3 = edge_index[0].astype(jnp.int32).reshape(NW, NCH, K)
    dst3 = edge_index[1].astype(jnp.int32).reshape(NW, NCH, K)
    zeros1 = jnp.zeros((SL,), jnp.float32)

    degp = pl.kernel(
        _deg_body,
        out_type=jax.ShapeDtypeStruct((2, NC, NPAD), jnp.float32),
        mesh=_mesh(),
        scratch_types=[
            pltpu.VMEM((NCH, K), jnp.int32),
            pltpu.VMEM((NCH, K), jnp.int32),
            pltpu.VMEM((K,), jnp.float32),
            pltpu.VMEM_SHARED((NPAD,), jnp.float32),
            pltpu.VMEM_SHARED((NPAD,), jnp.float32),
            pltpu.SemaphoreType.DMA,
        ],
    )(src3, dst3, zeros1)

    nb = N // RB
    feat_src = pl.pallas_call(
        _src_norm_body,
        grid=(RB,),
        in_specs=[
            pl.BlockSpec((nb, D), lambda i: (i, 0)),
            pl.BlockSpec((NC, nb, 1), lambda i: (0, i, 0)),
        ],
        out_specs=pl.BlockSpec((nb, D), lambda i: (i, 0)),
        out_shape=jax.ShapeDtypeStruct((N, D), jnp.float32),
    )(feat, degp[0][:, :, None])

    aggp = pl.kernel(
        _spmm_body,
        out_type=jax.ShapeDtypeStruct((NC, NPAD, D), jnp.float32),
        mesh=_mesh(),
        scratch_types=[
            pltpu.VMEM((NIB, K), jnp.int32),
            pltpu.VMEM((NIB, K), jnp.int32),
            pltpu.VMEM((NBUF, K, D), jnp.float32),
            pltpu.VMEM_SHARED((NPAD, D), jnp.float32),
            pltpu.SemaphoreType.DMA((NIB,)),
            pltpu.SemaphoreType.DMA((NBUF,)),
        ],
    )(feat_src, src3, dst3)

    out = pl.pallas_call(
        _out_body,
        grid=(RB,),
        in_specs=[
            pl.BlockSpec((NC, nb, D), lambda i: (0, i, 0)),
            pl.BlockSpec((D, D), lambda i: (0, 0)),
            pl.BlockSpec((NC, nb, 1), lambda i: (0, i, 0)),
            pl.BlockSpec((1, D), lambda i: (0, 0)),
        ],
        out_specs=pl.BlockSpec((nb, D), lambda i: (i, 0)),
        out_shape=jax.ShapeDtypeStruct((N, D), jnp.float32),
    )(aggp, weight, degp[1][:, :, None], bias.reshape(1, D))
    return out


# speedup vs baseline: 1.2073x; 1.0893x over previous
import jax
import jax.numpy as jnp
from jax import lax
from jax.experimental import pallas as pl
from jax.experimental.pallas import tpu as pltpu
from jax.experimental.pallas import tpu_sc as plsc

N = 10000
NPAD = 10240
E = 320000
D = 128
NC = 2
NS = 16
NW = NC * NS
EW = E // NW
K = 80
NCH = EW // K
SL = NPAD // NS
NBUF = 4
NIB = 2 * NBUF
NMAIN = (NCH // NIB) * NIB
AG = 5
RB = 10


def _mesh():
    return plsc.VectorSubcoreMesh(
        core_axis_name="c", subcore_axis_name="s", num_cores=NC, num_subcores=NS
    )


def _fill_ones(ones):
    for i in range(K // 16):
        ones[pl.ds(i * 16, 16)] = jnp.ones((16,), jnp.float32)


def _deg_body(src3, dst3, zeros1, degp, sidx, didx, ones, sh_out, sh_in, sem):
    c = lax.axis_index("c")
    s = lax.axis_index("s")
    wid = s * NC + c
    pltpu.sync_copy(zeros1, sh_out.at[pl.ds(s * SL, SL)])
    pltpu.sync_copy(zeros1, sh_in.at[pl.ds(s * SL, SL)])
    _fill_ones(ones)
    pltpu.sync_copy(src3.at[wid], sidx)
    pltpu.sync_copy(dst3.at[wid], didx)
    plsc.subcore_barrier()

    def group(g, carry):
        for t in range(AG):
            j = g * AG + t
            pltpu.async_copy(ones, sh_out.at[sidx.at[j]], sem, add=True)
            pltpu.async_copy(ones, sh_in.at[didx.at[j]], sem, add=True)
        for t in range(AG):
            j = g * AG + t
            pltpu.make_async_copy(ones, sh_out.at[sidx.at[j]], sem).wait()
            pltpu.make_async_copy(ones, sh_in.at[didx.at[j]], sem).wait()
        return carry

    lax.fori_loop(0, NCH // AG, group, 0)
    plsc.subcore_barrier()
    pltpu.sync_copy(sh_out.at[pl.ds(s * SL, SL)], degp.at[0, c, pl.ds(s * SL, SL)])
    pltpu.sync_copy(sh_in.at[pl.ds(s * SL, SL)], degp.at[1, c, pl.ds(s * SL, SL)])


def _spmm_body(table, src3, dst3, aggp, sbuf, dbuf, rows, sh_acc,
               isem, gsem):
    c = lax.axis_index("c")
    s = lax.axis_index("s")
    wid = s * NC + c
    zv = jnp.zeros((16,), jnp.float32)

    def zrow(i, carry):
        for kk in range(D // 16):
            rows[0, i, pl.ds(kk * 16, 16)] = zv
        return carry

    lax.fori_loop(0, K, zrow, 0)
    for r in range(SL // K):
        pltpu.sync_copy(rows.at[0], sh_acc.at[pl.ds(s * SL + r * K, K)])
    plsc.subcore_barrier()

    def fire_idx(j, slot):
        pltpu.async_copy(src3.at[wid, j], sbuf.at[slot], isem.at[slot])
        pltpu.async_copy(dst3.at[wid, j], dbuf.at[slot], isem.at[slot])

    def wait_idx(slot):
        pltpu.make_async_copy(src3.at[wid, 0], sbuf.at[slot], isem.at[slot]).wait()
        pltpu.make_async_copy(src3.at[wid, 0], dbuf.at[slot], isem.at[slot]).wait()

    def fire_gather(slot, b):
        pltpu.async_copy(table.at[sbuf.at[slot]], rows.at[b], gsem.at[b])

    def wait_gather(b):
        pltpu.make_async_copy(table.at[pl.ds(0, K)], rows.at[b], gsem.at[b]).wait()


    for b in range(NBUF):
        fire_idx(b, b)
    for b in range(NBUF):
        wait_idx(b)
        fire_gather(b, b)
        fire_idx(b + NBUF, b + NBUF)

    def outer(g, carry):
        for ib in range(NIB):
            b = ib % NBUF
            j = g * NIB + ib
            wait_gather(b)
            pltpu.sync_copy(rows.at[b], sh_acc.at[dbuf.at[ib]], add=True)

            @pl.when(j + NIB < NCH)
            def _refill():
                fire_idx(j + NIB, ib)

            nslot = (ib + NBUF) % NIB
            wait_idx(nslot)
            fire_gather(nslot, b)
        return carry

    lax.fori_loop(0, NMAIN // NIB, outer, 0)

    for t in range(NCH - NMAIN):
        j = NMAIN + t
        ib = j % NIB
        b = ib % NBUF
        wait_gather(b)
        pltpu.sync_copy(rows.at[b], sh_acc.at[dbuf.at[ib]], add=True)
        nxt = j + NBUF
        if nxt < NCH:
            nslot = nxt % NIB
            wait_idx(nslot)
            fire_gather(nslot, b)

    plsc.subcore_barrier()
    pltpu.sync_copy(sh_acc.at[pl.ds(s * SL, SL)], aggp.at[c, pl.ds(s * SL, SL)])


NG = N // D
NREM = N - NG * D


def _norm_t(d_ref):
    deg = d_ref[0] + d_ref[1]
    norm = lax.rsqrt(jnp.maximum(deg, 1.0))
    return jnp.transpose(norm)


def _src_norm_body(f_ref, d_ref, o_ref):
    nt = _norm_t(d_ref)
    for g in range(NG):
        o_ref[pl.ds(g * D, D), :] = f_ref[pl.ds(g * D, D), :] * nt[:, g:g + 1]
    o_ref[pl.ds(NG * D, NREM), :] = (
        f_ref[pl.ds(NG * D, NREM), :] * nt[0:NREM, NG:NG + 1]
    )


def _out_body(a_ref, w_ref, d_ref, b_ref, o_ref):
    agg = a_ref[0, 0:N, :] + a_ref[1, 0:N, :]
    rst = jnp.dot(agg, w_ref[...], preferred_element_type=jnp.float32)
    nt = _norm_t(d_ref)
    for g in range(NG):
        o_ref[pl.ds(g * D, D), :] = (
            rst[g * D:(g + 1) * D, :] * nt[:, g:g + 1] + b_ref[...]
        )
    o_ref[pl.ds(NG * D, NREM), :] = (
        rst[NG * D:N, :] * nt[0:NREM, NG:NG + 1] + b_ref[...]
    )


@jax.jit
def kernel(feat, edge_index, weight, bias):
    src3 = edge_index[0].astype(jnp.int32).reshape(NW, NCH, K)
    dst3 = edge_index[1].astype(jnp.int32).reshape(NW, NCH, K)
    zeros1 = jnp.zeros((SL,), jnp.float32)

    degp = pl.kernel(
        _deg_body,
        out_type=jax.ShapeDtypeStruct((2, NC, NPAD), jnp.float32),
        mesh=_mesh(),
        scratch_types=[
            pltpu.VMEM((NCH, K), jnp.int32),
            pltpu.VMEM((NCH, K), jnp.int32),
            pltpu.VMEM((K,), jnp.float32),
            pltpu.VMEM_SHARED((NPAD,), jnp.float32),
            pltpu.VMEM_SHARED((NPAD,), jnp.float32),
            pltpu.SemaphoreType.DMA,
        ],
    )(src3, dst3, zeros1)

    deg2 = degp.reshape(2, NC, NPAD // D, D)
    feat_src = pl.pallas_call(
        _src_norm_body,
        out_shape=jax.ShapeDtypeStruct((N, D), jnp.float32),
    )(feat, deg2[0])

    aggp = pl.kernel(
        _spmm_body,
        out_type=jax.ShapeDtypeStruct((NC, NPAD, D), jnp.float32),
        mesh=_mesh(),
        scratch_types=[
            pltpu.VMEM((NIB, K), jnp.int32),
            pltpu.VMEM((NIB, K), jnp.int32),
            pltpu.VMEM((NBUF, K, D), jnp.float32),
            pltpu.VMEM_SHARED((NPAD, D), jnp.float32),
            pltpu.SemaphoreType.DMA((NIB,)),
            pltpu.SemaphoreType.DMA((NBUF,)),
        ],
    )(feat_src, src3, dst3)

    out = pl.pallas_call(
        _out_body,
        out_shape=jax.ShapeDtypeStruct((N, D), jnp.float32),
    )(aggp, weight, deg2[1], bias.reshape(1, D))
    return out


# speedup vs baseline: 1.3286x; 1.1005x over previous
import jax
import jax.numpy as jnp
from jax import lax
from jax.experimental import pallas as pl
from jax.experimental.pallas import tpu as pltpu
from jax.experimental.pallas import tpu_sc as plsc

N = 10000
NPAD = 10240
E = 320000
D = 128
NC = 2
NS = 16
NW = NC * NS
EW = E // NW
K = 80
NCH = EW // K
SL = NPAD // NS
NBUF = 4
NIB = 2 * NBUF
NMAIN = (NCH // NIB) * NIB
AG = 5
RB = 10


def _mesh():
    return plsc.VectorSubcoreMesh(
        core_axis_name="c", subcore_axis_name="s", num_cores=NC, num_subcores=NS
    )


def _fill_ones(ones):
    for i in range(K // 16):
        ones[pl.ds(i * 16, 16)] = jnp.ones((16,), jnp.float32)


def _deg_body(e4, degp, sidx, didx, ones, zbuf, sh_out, sh_in, sem):
    c = lax.axis_index("c")
    s = lax.axis_index("s")
    wid = s * NC + c
    zv = jnp.zeros((16,), jnp.float32)

    def zstep(i, carry):
        zbuf[pl.ds(i * 16, 16)] = zv
        return carry

    lax.fori_loop(0, SL // 16, zstep, 0)
    pltpu.sync_copy(zbuf, sh_out.at[pl.ds(s * SL, SL)])
    pltpu.sync_copy(zbuf, sh_in.at[pl.ds(s * SL, SL)])
    _fill_ones(ones)
    pltpu.sync_copy(e4.at[0, wid], sidx)
    pltpu.sync_copy(e4.at[1, wid], didx)
    plsc.subcore_barrier()

    def group(g, carry):
        for t in range(AG):
            j = g * AG + t
            pltpu.async_copy(ones, sh_out.at[sidx.at[j]], sem, add=True)
            pltpu.async_copy(ones, sh_in.at[didx.at[j]], sem, add=True)
        for t in range(AG):
            j = g * AG + t
            pltpu.make_async_copy(ones, sh_out.at[sidx.at[j]], sem).wait()
            pltpu.make_async_copy(ones, sh_in.at[didx.at[j]], sem).wait()
        return carry

    lax.fori_loop(0, NCH // AG, group, 0)
    plsc.subcore_barrier()
    pltpu.sync_copy(sh_out.at[pl.ds(s * SL, SL)], degp.at[0, c, pl.ds(s * SL, SL)])
    pltpu.sync_copy(sh_in.at[pl.ds(s * SL, SL)], degp.at[1, c, pl.ds(s * SL, SL)])


def _spmm_body(table, e4, aggp, sbuf, dbuf, rows, sh_acc,
               isem, gsem):
    c = lax.axis_index("c")
    s = lax.axis_index("s")
    wid = s * NC + c
    zv = jnp.zeros((16,), jnp.float32)

    def zrow(i, carry):
        for kk in range(D // 16):
            rows[0, i, pl.ds(kk * 16, 16)] = zv
        return carry

    lax.fori_loop(0, K, zrow, 0)
    for r in range(SL // K):
        pltpu.sync_copy(rows.at[0], sh_acc.at[pl.ds(s * SL + r * K, K)])
    plsc.subcore_barrier()

    def fire_idx(j, slot):
        pltpu.async_copy(e4.at[0, wid, j], sbuf.at[slot], isem.at[slot])
        pltpu.async_copy(e4.at[1, wid, j], dbuf.at[slot], isem.at[slot])

    def wait_idx(slot):
        pltpu.make_async_copy(e4.at[0, wid, 0], sbuf.at[slot], isem.at[slot]).wait()
        pltpu.make_async_copy(e4.at[0, wid, 0], dbuf.at[slot], isem.at[slot]).wait()

    def fire_gather(slot, b):
        pltpu.async_copy(table.at[sbuf.at[slot]], rows.at[b], gsem.at[b])

    def wait_gather(b):
        pltpu.make_async_copy(table.at[pl.ds(0, K)], rows.at[b], gsem.at[b]).wait()


    for b in range(NBUF):
        fire_idx(b, b)
    for b in range(NBUF):
        wait_idx(b)
        fire_gather(b, b)
        fire_idx(b + NBUF, b + NBUF)

    def outer(g, carry):
        for ib in range(NIB):
            b = ib % NBUF
            j = g * NIB + ib
            wait_gather(b)
            pltpu.sync_copy(rows.at[b], sh_acc.at[dbuf.at[ib]], add=True)

            @pl.when(j + NIB < NCH)
            def _refill():
                fire_idx(j + NIB, ib)

            nslot = (ib + NBUF) % NIB
            wait_idx(nslot)
            fire_gather(nslot, b)
        return carry

    lax.fori_loop(0, NMAIN // NIB, outer, 0)

    for t in range(NCH - NMAIN):
        j = NMAIN + t
        ib = j % NIB
        b = ib % NBUF
        wait_gather(b)
        pltpu.sync_copy(rows.at[b], sh_acc.at[dbuf.at[ib]], add=True)
        nxt = j + NBUF
        if nxt < NCH:
            nslot = nxt % NIB
            wait_idx(nslot)
            fire_gather(nslot, b)

    plsc.subcore_barrier()
    pltpu.sync_copy(sh_acc.at[pl.ds(s * SL, SL)], aggp.at[c, pl.ds(s * SL, SL)])


NG = N // D
NREM = N - NG * D


def _norm_t(d_ref):
    deg = d_ref[0] + d_ref[1]
    norm = lax.rsqrt(jnp.maximum(deg, 1.0))
    return jnp.transpose(norm)


def _src_norm_body(f_ref, d_ref, o_ref):
    nt = _norm_t(d_ref)
    for g in range(NG):
        o_ref[pl.ds(g * D, D), :] = f_ref[pl.ds(g * D, D), :] * nt[:, g:g + 1]
    o_ref[pl.ds(NG * D, NREM), :] = (
        f_ref[pl.ds(NG * D, NREM), :] * nt[0:NREM, NG:NG + 1]
    )


def _out_body(a_ref, w_ref, d_ref, b_ref, o_ref):
    agg = a_ref[0, 0:N, :] + a_ref[1, 0:N, :]
    rst = jnp.dot(agg, w_ref[...], preferred_element_type=jnp.float32)
    nt = _norm_t(d_ref)
    for g in range(NG):
        o_ref[pl.ds(g * D, D), :] = (
            rst[g * D:(g + 1) * D, :] * nt[:, g:g + 1] + b_ref[...]
        )
    o_ref[pl.ds(NG * D, NREM), :] = (
        rst[NG * D:N, :] * nt[0:NREM, NG:NG + 1] + b_ref[...]
    )


@jax.jit
def kernel(feat, edge_index, weight, bias):
    e4 = edge_index.astype(jnp.int32).reshape(2, NW, NCH, K)

    degp = pl.kernel(
        _deg_body,
        out_type=jax.ShapeDtypeStruct((2, NC, NPAD), jnp.float32),
        mesh=_mesh(),
        scratch_types=[
            pltpu.VMEM((NCH, K), jnp.int32),
            pltpu.VMEM((NCH, K), jnp.int32),
            pltpu.VMEM((K,), jnp.float32),
            pltpu.VMEM((SL,), jnp.float32),
            pltpu.VMEM_SHARED((NPAD,), jnp.float32),
            pltpu.VMEM_SHARED((NPAD,), jnp.float32),
            pltpu.SemaphoreType.DMA,
        ],
    )(e4)

    deg2 = degp.reshape(2, NC, NPAD // D, D)
    feat_src = pl.pallas_call(
        _src_norm_body,
        out_shape=jax.ShapeDtypeStruct((N, D), jnp.float32),
    )(feat, deg2[0])

    aggp = pl.kernel(
        _spmm_body,
        out_type=jax.ShapeDtypeStruct((NC, NPAD, D), jnp.float32),
        mesh=_mesh(),
        scratch_types=[
            pltpu.VMEM((NIB, K), jnp.int32),
            pltpu.VMEM((NIB, K), jnp.int32),
            pltpu.VMEM((NBUF, K, D), jnp.float32),
            pltpu.VMEM_SHARED((NPAD, D), jnp.float32),
            pltpu.SemaphoreType.DMA((NIB,)),
            pltpu.SemaphoreType.DMA((NBUF,)),
        ],
    )(feat_src, e4)

    out = pl.pallas_call(
        _out_body,
        out_shape=jax.ShapeDtypeStruct((N, D), jnp.float32),
    )(aggp, weight, deg2[1], bias.reshape(1, D))
    return out
